# Initial kernel scaffold; baseline (speedup 1.0000x reference)
import jax
import jax.numpy as jnp
from jax.experimental import pallas as pl


def kernel(input_embeddings, feature_ids, feature_vals, feature_biases, bias):
    raise NotImplementedError("write your pallas kernel here")


# speedup vs baseline: 2.0708x; 2.0708x over previous
import functools

import jax
import jax.numpy as jnp
import numpy as np
from jax import lax
from jax.experimental import pallas as pl
from jax.experimental.pallas import tpu as pltpu
from jax.experimental.pallas import tpu_sc as plsc

B, F, D, V = 16384, 26, 16, 1000000
FD = F * D

NC, NS, L = 2, 16, 16
NW = NC * NS
PW = (B * F) // NW
RW = B // NW

@functools.cache
def _make_sc_bias_sum():
    mesh = plsc.VectorSubcoreMesh(
        core_axis_name="c", subcore_axis_name="s", num_cores=NC, num_subcores=NS
    )
    return pl.kernel(
        _sc_bias_sum_body,
        out_type=jax.ShapeDtypeStruct((B,), jnp.float32),
        mesh=mesh,
        scratch_types=[
            [pltpu.VMEM((RW,), jnp.int32) for _ in range(F)],
            [pltpu.VMEM((RW,), jnp.float32) for _ in range(F)],
            pltpu.VMEM((F, RW), jnp.float32),
            pltpu.VMEM((RW,), jnp.float32),
            pltpu.SemaphoreType.DMA,
        ],
    )


def _sc_bias_sum_body(table_hbm, ids_hbm, vals_hbm, out_hbm, idx_fs, g_fs,
                      vals_v, out_v, sem):
    i32 = jnp.int32
    wid = lax.axis_index("s") * i32(NC) + lax.axis_index("c")
    rbase = wid * i32(RW)

    descs = []
    for f in range(F):
        pltpu.sync_copy(ids_hbm.at[i32(f), pl.ds(rbase, RW)], idx_fs[f])
        descs.append(pltpu.async_copy(table_hbm.at[idx_fs[f]], g_fs[f], sem))
    pltpu.sync_copy(vals_hbm.at[:, pl.ds(rbase, RW)], vals_v)
    for d in descs:
        d.wait()

    def row_group(i, _):
        off = i * i32(L)
        acc = g_fs[0][pl.ds(off, L)] * vals_v[i32(0), pl.ds(off, L)]
        for f in range(1, F):
            acc = acc + g_fs[f][pl.ds(off, L)] * vals_v[i32(f), pl.ds(off, L)]
        out_v[pl.ds(off, L)] = acc
        return i32(0)

    lax.fori_loop(i32(0), i32(RW // L), row_group, i32(0))
    pltpu.sync_copy(out_v, out_hbm.at[pl.ds(rbase, RW)])


_BB = 2048


def _tc_body(x_ref, m_ref, o_ref):
    x = x_ref[...]
    m = m_ref[...]
    s = jnp.dot(x, m, preferred_element_type=jnp.float32)
    t1 = jnp.sum(s * s, axis=1)
    t2 = jnp.sum(x * x, axis=1)
    o_ref[...] = (t1 - t2) * (1.0 / (2.0 * D))


_tc_call = pl.pallas_call(
    _tc_body,
    out_shape=jax.ShapeDtypeStruct((B,), jnp.float32),
    grid=(B // _BB,),
    in_specs=[
        pl.BlockSpec((_BB, FD), lambda i: (i, jnp.int32(0))),
        pl.BlockSpec((FD, D), lambda i: (jnp.int32(0), jnp.int32(0))),
    ],
    out_specs=pl.BlockSpec((_BB,), lambda i: (i,)),
)

_M_np = np.zeros((FD, D), dtype=np.float32)
_M_np[np.arange(FD), np.arange(FD) % D] = 1.0


@jax.jit
def kernel(input_embeddings, feature_ids, feature_vals, feature_biases, bias):
    x = input_embeddings.reshape(B, FD)
    m = jnp.asarray(_M_np)
    pred2 = _tc_call(x, m)

    ids = feature_ids.T.astype(jnp.int32)
    vals = feature_vals.T.astype(jnp.float32)
    table = feature_biases.reshape(V)
    bias_sums = _make_sc_bias_sum()(table, ids, vals)

    return pred2 + bias_sums + bias[0]


# speedup vs baseline: 2.1272x; 1.0272x over previous
import functools

import jax
import jax.numpy as jnp
import numpy as np
from jax import lax
from jax.experimental import pallas as pl
from jax.experimental.pallas import tpu as pltpu
from jax.experimental.pallas import tpu_sc as plsc

B, F, D, V = 16384, 26, 16, 1000000
FD = F * D

NC, NS, L = 2, 16, 16
NW = NC * NS
PW = (B * F) // NW
RW = B // NW

@functools.cache
def _make_sc_bias_sum():
    mesh = plsc.VectorSubcoreMesh(
        core_axis_name="c", subcore_axis_name="s", num_cores=NC, num_subcores=NS
    )
    return pl.kernel(
        _sc_bias_sum_body,
        out_type=jax.ShapeDtypeStruct((B,), jnp.float32),
        mesh=mesh,
        scratch_types=[
            [pltpu.VMEM((RW,), jnp.int32) for _ in range(F)],
            [pltpu.VMEM((RW,), jnp.float32) for _ in range(F)],
            pltpu.VMEM((F, RW), jnp.float32),
            pltpu.VMEM((RW,), jnp.float32),
            pltpu.VMEM_SHARED((V,), jnp.float32),
            pltpu.VMEM((8192,), jnp.float32),
            pltpu.SemaphoreType.DMA,
        ],
    )


_TCH = 62504


def _sc_bias_sum_body(table_hbm, ids_hbm, vals_hbm, out_hbm, idx_fs, g_fs,
                      vals_v, out_v, table_sh, bounce_v, sem):
    i32 = jnp.int32
    sid = lax.axis_index("s")
    wid = sid * i32(NC) + lax.axis_index("c")
    rbase = wid * i32(RW)

    toff = jnp.minimum(sid * i32(_TCH), i32(V - _TCH))
    sub_off = 0
    for sz in [8192] * 7 + [_TCH - 7 * 8192]:
        src = table_hbm.at[pl.ds(toff + i32(sub_off), sz)]
        pltpu.sync_copy(src, bounce_v.at[pl.ds(i32(0), sz)])
        pltpu.sync_copy(bounce_v.at[pl.ds(i32(0), sz)],
                        table_sh.at[pl.ds(toff + i32(sub_off), sz)])
        sub_off += sz
    for f in range(F):
        pltpu.sync_copy(ids_hbm.at[i32(f), pl.ds(rbase, RW)], idx_fs[f])
    pltpu.sync_copy(vals_hbm.at[:, pl.ds(rbase, RW)], vals_v)
    plsc.subcore_barrier()

    descs = [
        pltpu.async_copy(table_sh.at[idx_fs[f]], g_fs[f], sem)
        for f in range(F)
    ]
    for d in descs:
        d.wait()

    def row_group(i, _):
        off = i * i32(L)
        acc = g_fs[0][pl.ds(off, L)] * vals_v[i32(0), pl.ds(off, L)]
        for f in range(1, F):
            acc = acc + g_fs[f][pl.ds(off, L)] * vals_v[i32(f), pl.ds(off, L)]
        out_v[pl.ds(off, L)] = acc
        return i32(0)

    lax.fori_loop(i32(0), i32(RW // L), row_group, i32(0))
    pltpu.sync_copy(out_v, out_hbm.at[pl.ds(rbase, RW)])


_BB = 2048


def _tc_body(x_ref, m_ref, o_ref):
    x = x_ref[...]
    m = m_ref[...]
    s = jnp.dot(x, m, preferred_element_type=jnp.float32)
    t1 = jnp.sum(s * s, axis=1)
    t2 = jnp.sum(x * x, axis=1)
    o_ref[...] = (t1 - t2) * (1.0 / (2.0 * D))


_tc_call = pl.pallas_call(
    _tc_body,
    out_shape=jax.ShapeDtypeStruct((B,), jnp.float32),
    grid=(B // _BB,),
    in_specs=[
        pl.BlockSpec((_BB, FD), lambda i: (i, jnp.int32(0))),
        pl.BlockSpec((FD, D), lambda i: (jnp.int32(0), jnp.int32(0))),
    ],
    out_specs=pl.BlockSpec((_BB,), lambda i: (i,)),
)

_M_np = np.zeros((FD, D), dtype=np.float32)
_M_np[np.arange(FD), np.arange(FD) % D] = 1.0


@jax.jit
def kernel(input_embeddings, feature_ids, feature_vals, feature_biases, bias):
    x = input_embeddings.reshape(B, FD)
    m = jnp.asarray(_M_np)
    pred2 = _tc_call(x, m)

    ids = feature_ids.T.astype(jnp.int32)
    vals = feature_vals.T.astype(jnp.float32)
    table = feature_biases.reshape(V)
    bias_sums = _make_sc_bias_sum()(table, ids, vals)

    return pred2 + bias_sums + bias[0]
